# baseline (speedup 1.0000x reference)
import functools

import jax
import jax.numpy as jnp
from jax import lax
from jax.experimental import pallas as pl
from jax.experimental.pallas import tpu as pltpu

DIM = 16
LANE = 128


def _glove_body(i_ref, j_ref, wblk, wtblk, bblk, btblk, out):
    ci = i_ref[0] % LANE
    cj = j_ref[0] % LANE
    lane2 = lax.broadcasted_iota(jnp.int32, (DIM, LANE), 1)
    lane1 = lax.broadcasted_iota(jnp.int32, (1, LANE), 1)
    zero2 = jnp.zeros((DIM, LANE), jnp.float32)
    zero1 = jnp.zeros((1, LANE), jnp.float32)
    wi = jnp.sum(jnp.where(lane2 == ci, wblk[...], zero2), axis=1)
    wj = jnp.sum(jnp.where(lane2 == cj, wtblk[...], zero2), axis=1)
    dot = jnp.sum(wi * wj)
    bi = jnp.sum(jnp.where(lane1 == ci, bblk[...], zero1))
    bj = jnp.sum(jnp.where(lane1 == cj, btblk[...], zero1))
    out[0, 0] = dot + bi + bj


@jax.jit
def _glove_call(i1, j1, WT, WtT, bT, btT):
    grid_spec = pltpu.PrefetchScalarGridSpec(
        num_scalar_prefetch=2,
        grid=(1,),
        in_specs=[
            pl.BlockSpec((DIM, LANE), lambda g, si, sj: (0, si[0] // LANE)),
            pl.BlockSpec((DIM, LANE), lambda g, si, sj: (0, sj[0] // LANE)),
            pl.BlockSpec((1, LANE), lambda g, si, sj: (0, si[0] // LANE)),
            pl.BlockSpec((1, LANE), lambda g, si, sj: (0, sj[0] // LANE)),
        ],
        out_specs=pl.BlockSpec(
            (1, 1), lambda g, si, sj: (0, 0), memory_space=pltpu.SMEM),
    )
    fn = pl.pallas_call(
        _glove_body,
        grid_spec=grid_spec,
        out_shape=jax.ShapeDtypeStruct((1, 1), jnp.float32),
    )
    return fn(i1, j1, WT, WtT, bT, btT)


def kernel(i, j, W, W_tilde, b, b_tilde):
    i1 = jnp.reshape(i, (1,)).astype(jnp.int32)
    j1 = jnp.reshape(j, (1,)).astype(jnp.int32)
    out = _glove_call(i1, j1, W.T, W_tilde.T, b.T, b_tilde.T)
    return out[0, 0]


# speedup vs baseline: 1.0387x; 1.0387x over previous
import jax
import jax.numpy as jnp
from jax import lax
from jax.experimental import pallas as pl
from jax.experimental.pallas import tpu as pltpu

DIM = 16
LANE = 128


def _glove_body(i_ref, j_ref, wblk, wtblk, bblk, btblk, out):
    ci = i_ref[0] % LANE
    cj = j_ref[0] % LANE
    lane2 = lax.broadcasted_iota(jnp.int32, (DIM, LANE), 1)
    lane1 = lax.broadcasted_iota(jnp.int32, (1, LANE), 1)
    zero2 = jnp.zeros((DIM, LANE), jnp.float32)
    zero1 = jnp.zeros((1, LANE), jnp.float32)
    wi = jnp.sum(jnp.where(lane2 == ci, wblk[...], zero2), axis=1)
    wj = jnp.sum(jnp.where(lane2 == cj, wtblk[...], zero2), axis=1)
    dot = jnp.sum(wi * wj)
    bi = jnp.sum(jnp.where(lane1 == ci, bblk[...], zero1))
    bj = jnp.sum(jnp.where(lane1 == cj, btblk[...], zero1))
    out[0, 0] = dot + bi + bj


@jax.jit
def _glove_call(i1, j1, WT, WtT, bT, btT):
    grid_spec = pltpu.PrefetchScalarGridSpec(
        num_scalar_prefetch=2,
        grid=(1,),
        in_specs=[
            pl.BlockSpec((DIM, LANE), lambda g, si, sj: (0, si[0] // LANE)),
            pl.BlockSpec((DIM, LANE), lambda g, si, sj: (0, sj[0] // LANE)),
            pl.BlockSpec((1, LANE), lambda g, si, sj: (0, si[0] // LANE)),
            pl.BlockSpec((1, LANE), lambda g, si, sj: (0, sj[0] // LANE)),
        ],
        out_specs=pl.BlockSpec(
            (1, 1), lambda g, si, sj: (0, 0), memory_space=pltpu.SMEM),
    )
    fn = pl.pallas_call(
        _glove_body,
        grid_spec=grid_spec,
        out_shape=jax.ShapeDtypeStruct((1, 1), jnp.float32),
    )
    return fn(i1, j1, WT, WtT, bT, btT)


def kernel(i, j, W, W_tilde, b, b_tilde):
    i1 = jnp.reshape(i, (1,)).astype(jnp.int32)
    j1 = jnp.reshape(j, (1,)).astype(jnp.int32)
    out = _glove_call(i1, j1, W.T, W_tilde.T, b.T, b_tilde.T)
    return out[0, 0]
